# baseline (speedup 1.0000x reference)
import math

import jax
import jax.numpy as jnp
from jax import lax
from jax.experimental import pallas as pl
from jax.experimental.pallas import tpu as pltpu

_N = 1000
_C = 128
_HW = 50
_OUT = 7
_SCALE = 1.0 / 16.0
_NCLS = 91
_REP = 1024
_SCORE_TH = 0.05
_NMS_TH = 0.5
_DETS = 100
_IMG = 800.0
_LOGMAX = math.log(1000.0 / 16.0)

_PB = 8
_RB = _PB * _OUT * _OUT
_NQ = _N * (_NCLS - 1)
_NQP = 90112
_QROWS = _NQP // 128


def _roi_body(props_ref, feat_ref, out_ref):
    r_p = lax.broadcasted_iota(jnp.int32, (_RB, 1), 0)
    p_loc = r_p // 49
    ij = r_p % 49
    i_ = (ij // _OUT).astype(jnp.float32)
    j_ = (ij % _OUT).astype(jnp.float32)
    onehot = (p_loc == lax.broadcasted_iota(jnp.int32, (_RB, _PB), 1)).astype(
        jnp.float32)
    pr = jnp.dot(onehot, props_ref[...], preferred_element_type=jnp.float32, precision=lax.Precision.HIGHEST)
    x1 = pr[:, 0:1] * _SCALE
    y1 = pr[:, 1:2] * _SCALE
    x2 = pr[:, 2:3] * _SCALE
    y2 = pr[:, 3:4] * _SCALE
    bw = jnp.maximum(x2 - x1, 1.0) / _OUT
    bh = jnp.maximum(y2 - y1, 1.0) / _OUT
    gx = jnp.clip(x1 + bw * (j_ + 0.5), 0.0, _HW - 1.0)
    gy = jnp.clip(y1 + bh * (i_ + 0.5), 0.0, _HW - 1.0)
    x0 = jnp.floor(gx)
    lx = gx - x0
    x1i = jnp.minimum(x0 + 1.0, _HW - 1.0)
    y0 = jnp.floor(gy)
    ly = gy - y0
    y1i = jnp.minimum(y0 + 1.0, _HW - 1.0)
    q = lax.broadcasted_iota(jnp.int32, (1, _HW * _HW), 1)
    cy = (q // _HW).astype(jnp.float32)
    cx = (q % _HW).astype(jnp.float32)
    eqy0 = cy == y0
    eqy1 = cy == y1i
    eqx0 = cx == x0
    eqx1 = cx == x1i
    feat = feat_ref[...]

    def gat(m):
        return jnp.dot(m.astype(jnp.float32), feat,
                       preferred_element_type=jnp.float32,
                       precision=lax.Precision.HIGHEST)

    f00 = gat(eqy0 & eqx0)
    f01 = gat(eqy0 & eqx1)
    f10 = gat(eqy1 & eqx0)
    f11 = gat(eqy1 & eqx1)
    out_ref[...] = (f00 * (1 - ly) * (1 - lx) + f01 * (1 - ly) * lx
                    + f10 * ly * (1 - lx) + f11 * ly * lx)


def _fc1_body(x_ref, w_ref, b_ref, out_ref):
    out_ref[...] = jnp.maximum(
        jnp.dot(x_ref[...], w_ref[...], preferred_element_type=jnp.float32)
        + b_ref[...], 0.0)


def _head_body(h1_ref, w2_ref, b2_ref, wc_ref, bc_ref, wr_ref, br_ref,
               props_ref, sc_ref, bx1_ref, by1_ref, bx2_ref, by2_ref):
    h2 = jnp.maximum(
        jnp.dot(h1_ref[...], w2_ref[...], preferred_element_type=jnp.float32)
        + b2_ref[...], 0.0)
    logits = jnp.dot(h2, wc_ref[...],
                     preferred_element_type=jnp.float32) + bc_ref[...]
    col = lax.broadcasted_iota(jnp.int32, (_N, 128), 1)
    logits = jnp.where(col < _NCLS, logits, -1e30)
    m = jnp.max(logits, axis=1, keepdims=True)
    e = jnp.exp(logits - m)
    sc_ref[...] = e / jnp.sum(e, axis=1, keepdims=True)

    reg = jnp.dot(h2, wr_ref[...],
                  preferred_element_type=jnp.float32) + br_ref[...]
    px1 = props_ref[:, 0:1]
    py1 = props_ref[:, 1:2]
    px2 = props_ref[:, 2:3]
    py2 = props_ref[:, 3:4]
    wp = px2 - px1
    hp = py2 - py1
    cxp = px1 + 0.5 * wp
    cyp = py1 + 0.5 * hp
    dx = reg[:, 0:128] / 10.0
    dy = reg[:, 128:256] / 10.0
    dw = jnp.minimum(reg[:, 256:384] / 5.0, _LOGMAX)
    dh = jnp.minimum(reg[:, 384:512] / 5.0, _LOGMAX)
    pcx = dx * wp + cxp
    pcy = dy * hp + cyp
    pw = jnp.exp(dw) * wp
    ph = jnp.exp(dh) * hp
    bx1_ref[...] = jnp.clip(pcx - 0.5 * pw, 0.0, _IMG)
    by1_ref[...] = jnp.clip(pcy - 0.5 * ph, 0.0, _IMG)
    bx2_ref[...] = jnp.clip(pcx + 0.5 * pw, 0.0, _IMG)
    by2_ref[...] = jnp.clip(pcy + 0.5 * ph, 0.0, _IMG)


def _nms_body(fs_ref, x1_ref, y1_ref, x2_ref, y2_ref, sel_ref,
              s_ref, ox1_ref, oy1_ref, ox2_ref, oy2_ref, aa_ref, done_ref):
    rq = lax.broadcasted_iota(jnp.int32, (_QROWS, 128), 0)
    cq = lax.broadcasted_iota(jnp.int32, (_QROWS, 128), 1)
    q = rq * 128 + cq
    lbl = (q % (_NCLS - 1) + 1).astype(jnp.float32)
    off = lbl * (_IMG + 1.0)
    x1 = x1_ref[...]
    y1 = y1_ref[...]
    x2 = x2_ref[...]
    y2 = y2_ref[...]
    fs = fs_ref[...]
    valid = ((fs > _SCORE_TH) & (x2 - x1 >= 0.01) & (y2 - y1 >= 0.01)
             & (q < _NQ))
    s_ref[...] = jnp.where(valid, fs, -1e30)
    ox1_ref[...] = x1 + off
    oy1_ref[...] = y1 + off
    ox2_ref[...] = x2 + off
    oy2_ref[...] = y2 + off
    aa_ref[...] = (ox2_ref[...] - ox1_ref[...]) * (oy2_ref[...] - oy1_ref[...])
    sel_ref[...] = jnp.zeros((128, 128), jnp.int32)
    done_ref[0] = 0

    def body(it, carry):
        @pl.when(done_ref[0] == 0)
        def _():
            s = s_ref[...]
            mval = jnp.max(s)
            jq = jnp.min(jnp.where(s == mval, q, jnp.int32(2**30)))
            onehot = q == jq
            b0x1 = jnp.sum(jnp.where(onehot, ox1_ref[...], 0.0))
            b0y1 = jnp.sum(jnp.where(onehot, oy1_ref[...], 0.0))
            b0x2 = jnp.sum(jnp.where(onehot, ox2_ref[...], 0.0))
            b0y2 = jnp.sum(jnp.where(onehot, oy2_ref[...], 0.0))
            lane = lax.broadcasted_iota(jnp.int32, (1, 128), 1)
            row = jnp.where(lane == 0, jq,
                            jnp.where(lane == 1, jnp.int32(1), 0))
            sel_ref[pl.ds(it, 1), :] = row
            xx1 = jnp.maximum(b0x1, ox1_ref[...])
            yy1 = jnp.maximum(b0y1, oy1_ref[...])
            xx2 = jnp.minimum(b0x2, ox2_ref[...])
            yy2 = jnp.minimum(b0y2, oy2_ref[...])
            inter = (jnp.maximum(xx2 - xx1, 0.0) * jnp.maximum(yy2 - yy1, 0.0))
            a0 = (b0x2 - b0x1) * (b0y2 - b0y1)
            iou = inter / (a0 + aa_ref[...] - inter + 1e-9)
            s_new = jnp.where(iou <= _NMS_TH, s, -1e30)
            s_ref[...] = s_new
            done_ref[0] = jnp.where(jnp.max(s_new) > -1e29, 0, 1)
        return carry

    lax.fori_loop(0, _DETS, body, 0)


def kernel(features, proposals, W_fc1, b_fc1, W_fc2, b_fc2, W_cls, b_cls,
           W_reg, b_reg):
    f32 = jnp.float32
    featYX = jnp.transpose(features, (1, 2, 0)).reshape(_HW * _HW, _C)
    Wc_pad = jnp.zeros((_REP, 128), f32).at[:, :_NCLS].set(W_cls)
    bc_pad = jnp.zeros((1, 128), f32).at[0, :_NCLS].set(b_cls)
    cls_idx = jnp.arange(_NCLS)
    Wr_pad = jnp.zeros((_REP, 512), f32)
    br_pad = jnp.zeros((1, 512), f32)
    for g in range(4):
        Wr_pad = Wr_pad.at[:, g * 128 + cls_idx].set(W_reg[:, g::4])
        br_pad = br_pad.at[0, g * 128 + cls_idx].set(b_reg[g::4])

    rf = pl.pallas_call(
        _roi_body,
        grid=(_N // _PB,),
        in_specs=[
            pl.BlockSpec((_PB, 4), lambda b: (b, 0)),
            pl.BlockSpec((_HW * _HW, _C), lambda b: (0, 0)),
        ],
        out_specs=pl.BlockSpec((_RB, _C), lambda b: (b, 0)),
        out_shape=jax.ShapeDtypeStruct((_N * _OUT * _OUT, _C), f32),
        compiler_params=pltpu.CompilerParams(
            dimension_semantics=("arbitrary",)),
    )(proposals, featYX)
    rf2 = rf.reshape(_N, _OUT * _OUT, _C).transpose(0, 2, 1).reshape(
        _N, _OUT * _OUT * _C)

    NB = 256
    h1 = pl.pallas_call(
        _fc1_body,
        grid=(_REP // NB,),
        in_specs=[
            pl.BlockSpec((_N, _OUT * _OUT * _C), lambda n: (0, 0)),
            pl.BlockSpec((_OUT * _OUT * _C, NB), lambda n: (0, n)),
            pl.BlockSpec((1, NB), lambda n: (0, n)),
        ],
        out_specs=pl.BlockSpec((_N, NB), lambda n: (0, n)),
        out_shape=jax.ShapeDtypeStruct((_N, _REP), f32),
        compiler_params=pltpu.CompilerParams(
            dimension_semantics=("arbitrary",)),
    )(rf2, W_fc1, b_fc1.reshape(1, _REP))

    sc, bx1, by1, bx2, by2 = pl.pallas_call(
        _head_body,
        in_specs=[
            pl.BlockSpec((_N, _REP), lambda: (0, 0)),
            pl.BlockSpec((_REP, _REP), lambda: (0, 0)),
            pl.BlockSpec((1, _REP), lambda: (0, 0)),
            pl.BlockSpec((_REP, 128), lambda: (0, 0)),
            pl.BlockSpec((1, 128), lambda: (0, 0)),
            pl.BlockSpec((_REP, 512), lambda: (0, 0)),
            pl.BlockSpec((1, 512), lambda: (0, 0)),
            pl.BlockSpec((_N, 4), lambda: (0, 0)),
        ],
        out_specs=[pl.BlockSpec((_N, 128), lambda: (0, 0))] * 5,
        out_shape=[jax.ShapeDtypeStruct((_N, 128), f32)] * 5,
    )(h1, W_fc2, b_fc2.reshape(1, _REP), Wc_pad, bc_pad, Wr_pad, br_pad,
      proposals)

    def flat_pad(a, padval):
        v = a[:, 1:_NCLS].reshape(-1)
        v = jnp.concatenate([v, jnp.full((_NQP - _NQ,), padval, f32)])
        return v.reshape(_QROWS, 128)

    fsq = flat_pad(sc, 0.0)
    fx1 = flat_pad(bx1, 0.0)
    fy1 = flat_pad(by1, 0.0)
    fx2 = flat_pad(bx2, 0.0)
    fy2 = flat_pad(by2, 0.0)

    sel = pl.pallas_call(
        _nms_body,
        in_specs=[pl.BlockSpec((_QROWS, 128), lambda: (0, 0))] * 5,
        out_specs=pl.BlockSpec((128, 128), lambda: (0, 0)),
        out_shape=jax.ShapeDtypeStruct((128, 128), jnp.int32),
        scratch_shapes=[
            pltpu.VMEM((_QROWS, 128), f32),
            pltpu.VMEM((_QROWS, 128), f32),
            pltpu.VMEM((_QROWS, 128), f32),
            pltpu.VMEM((_QROWS, 128), f32),
            pltpu.VMEM((_QROWS, 128), f32),
            pltpu.VMEM((_QROWS, 128), f32),
            pltpu.SMEM((1,), jnp.int32),
        ],
    )(fsq, fx1, fy1, fx2, fy2)

    fin = sel[:_DETS, 0]
    finv = sel[:_DETS, 1] > 0
    fb = jnp.stack([fx1.reshape(-1), fy1.reshape(-1),
                    fx2.reshape(-1), fy2.reshape(-1)], axis=1)
    fs = fsq.reshape(-1)
    boxes = jnp.where(finv[:, None], fb[fin], 0.0)
    scores = jnp.where(finv, fs[fin], 0.0)
    labels = jnp.where(finv, (fin % (_NCLS - 1) + 1).astype(jnp.int32), 0)
    return boxes, scores, labels


# speedup vs baseline: 3.9143x; 3.9143x over previous
import functools
import math

import jax
import jax.numpy as jnp
from jax import lax
from jax.experimental import pallas as pl
from jax.experimental.pallas import tpu as pltpu
from jax.experimental.pallas import tpu_sc as plsc

_N = 1000
_C = 128
_HW = 50
_OUT = 7
_SCALE = 1.0 / 16.0
_NCLS = 91
_REP = 1024
_SCORE_TH = 0.05
_NMS_TH = 0.5
_DETS = 100
_IMG = 800.0
_LOGMAX = math.log(1000.0 / 16.0)

_PB = 8
_RB = _PB * _OUT * _OUT
_NQ = _N * (_NCLS - 1)
_NQP = 90112
_QROWS = _NQP // 128


def _roi_body(props_ref, feat_ref, out_ref):
    r_p = lax.broadcasted_iota(jnp.int32, (_RB, 1), 0)
    p_loc = r_p // 49
    ij = r_p % 49
    i_ = (ij // _OUT).astype(jnp.float32)
    j_ = (ij % _OUT).astype(jnp.float32)
    onehot = (p_loc == lax.broadcasted_iota(jnp.int32, (_RB, _PB), 1)).astype(
        jnp.float32)
    pr = jnp.dot(onehot, props_ref[...], preferred_element_type=jnp.float32, precision=lax.Precision.HIGHEST)
    x1 = pr[:, 0:1] * _SCALE
    y1 = pr[:, 1:2] * _SCALE
    x2 = pr[:, 2:3] * _SCALE
    y2 = pr[:, 3:4] * _SCALE
    bw = jnp.maximum(x2 - x1, 1.0) / _OUT
    bh = jnp.maximum(y2 - y1, 1.0) / _OUT
    gx = jnp.clip(x1 + bw * (j_ + 0.5), 0.0, _HW - 1.0)
    gy = jnp.clip(y1 + bh * (i_ + 0.5), 0.0, _HW - 1.0)
    x0 = jnp.floor(gx)
    lx = gx - x0
    x1i = jnp.minimum(x0 + 1.0, _HW - 1.0)
    y0 = jnp.floor(gy)
    ly = gy - y0
    y1i = jnp.minimum(y0 + 1.0, _HW - 1.0)
    q = lax.broadcasted_iota(jnp.int32, (1, _HW * _HW), 1)
    cy = (q // _HW).astype(jnp.float32)
    cx = (q % _HW).astype(jnp.float32)
    eqy0 = cy == y0
    eqy1 = cy == y1i
    eqx0 = cx == x0
    eqx1 = cx == x1i
    feat = feat_ref[...]

    def gat(m):
        return jnp.dot(m.astype(jnp.float32), feat,
                       preferred_element_type=jnp.float32,
                       precision=lax.Precision.HIGHEST)

    f00 = gat(eqy0 & eqx0)
    f01 = gat(eqy0 & eqx1)
    f10 = gat(eqy1 & eqx0)
    f11 = gat(eqy1 & eqx1)
    out_ref[...] = (f00 * (1 - ly) * (1 - lx) + f01 * (1 - ly) * lx
                    + f10 * ly * (1 - lx) + f11 * ly * lx)


_NW = 32
_TPW = 1568
_CH = 112
_NCH = _TPW // _CH


def _roi_sc_body(props_hbm, feat_hbm, out_hbm, x1v, y1v, x2v, y2v,
                 i00, i01, i10, i11, lyv, lxv,
                 f00, f01, f10, f11, outv, sem):
    wid = lax.axis_index("s") * 2 + lax.axis_index("c")
    pltpu.sync_copy(props_hbm.at[0], x1v)
    pltpu.sync_copy(props_hbm.at[1], y1v)
    pltpu.sync_copy(props_hbm.at[2], x2v)
    pltpu.sync_copy(props_hbm.at[3], y2v)
    lanes = lax.broadcasted_iota(jnp.int32, (16,), 0)
    zc = jnp.zeros((16,), jnp.int32)

    def chunk_body(ch, _):
        base = wid * _TPW + ch * _CH

        def param_body(s, _):
            t = base + s * 16 + lanes
            tf = t.astype(jnp.float32) + 0.5
            p = jnp.minimum((tf * (1.0 / 49.0)).astype(jnp.int32), _N - 1)
            ij = t - p * 49
            ijf = ij.astype(jnp.float32) + 0.5
            iq = (ijf * (1.0 / 7.0)).astype(jnp.int32)
            i_ = iq.astype(jnp.float32)
            j_ = (ij - iq * 7).astype(jnp.float32)
            x1 = plsc.load_gather(x1v, [p]) * _SCALE
            y1 = plsc.load_gather(y1v, [p]) * _SCALE
            x2 = plsc.load_gather(x2v, [p]) * _SCALE
            y2 = plsc.load_gather(y2v, [p]) * _SCALE
            bw = jnp.maximum(x2 - x1, 1.0) / _OUT
            bh = jnp.maximum(y2 - y1, 1.0) / _OUT
            gx = jnp.clip(x1 + bw * (j_ + 0.5), 0.0, _HW - 1.0)
            gy = jnp.clip(y1 + bh * (i_ + 0.5), 0.0, _HW - 1.0)
            x0 = gx.astype(jnp.int32)
            y0 = gy.astype(jnp.int32)
            lx = gx - x0.astype(jnp.float32)
            ly = gy - y0.astype(jnp.float32)
            x1i = jnp.minimum(x0 + 1, _HW - 1)
            y1i = jnp.minimum(y0 + 1, _HW - 1)
            sl = pl.ds(s * 16, 16)
            i00[sl] = y0 * _HW + x0
            i01[sl] = y0 * _HW + x1i
            i10[sl] = y1i * _HW + x0
            i11[sl] = y1i * _HW + x1i
            lyv[sl] = ly
            lxv[sl] = lx
            return 0

        lax.fori_loop(0, _CH // 16, param_body, 0)
        cp0 = pltpu.async_copy(feat_hbm.at[i00], f00, sem)
        cp1 = pltpu.async_copy(feat_hbm.at[i01], f01, sem)
        cp2 = pltpu.async_copy(feat_hbm.at[i10], f10, sem)
        cp3 = pltpu.async_copy(feat_hbm.at[i11], f11, sem)
        cp0.wait()
        cp1.wait()
        cp2.wait()
        cp3.wait()

        def pt_body(pt, _):
            ib = zc + pt
            ly = plsc.load_gather(lyv, [ib])
            lx = plsc.load_gather(lxv, [ib])
            a = 1.0 - ly
            b = 1.0 - lx
            for cs in range(8):
                sl = pl.ds(cs * 16, 16)
                v = (f00[pt, sl] * a * b + f01[pt, sl] * a * lx
                     + f10[pt, sl] * ly * b + f11[pt, sl] * ly * lx)
                outv[pt, sl] = v
            return 0

        lax.fori_loop(0, _CH, pt_body, 0)
        pltpu.sync_copy(outv, out_hbm.at[pl.ds(base, _CH)])
        return 0

    lax.fori_loop(0, _NCH, chunk_body, 0)


def _roi_sc(proposals, featYX):
    f32 = jnp.float32
    kern = functools.partial(
        pl.kernel,
        out_type=jax.ShapeDtypeStruct((_NW * _TPW, _C), f32),
        mesh=plsc.VectorSubcoreMesh(core_axis_name="c", subcore_axis_name="s"),
        compiler_params=pltpu.CompilerParams(needs_layout_passes=False),
        scratch_types=[
            pltpu.VMEM((1024,), f32),
            pltpu.VMEM((1024,), f32),
            pltpu.VMEM((1024,), f32),
            pltpu.VMEM((1024,), f32),
            pltpu.VMEM((_CH,), jnp.int32),
            pltpu.VMEM((_CH,), jnp.int32),
            pltpu.VMEM((_CH,), jnp.int32),
            pltpu.VMEM((_CH,), jnp.int32),
            pltpu.VMEM((_CH,), f32),
            pltpu.VMEM((_CH,), f32),
            pltpu.VMEM((_CH, _C), f32),
            pltpu.VMEM((_CH, _C), f32),
            pltpu.VMEM((_CH, _C), f32),
            pltpu.VMEM((_CH, _C), f32),
            pltpu.VMEM((_CH, _C), f32),
            pltpu.SemaphoreType.DMA,
        ],
    )(_roi_sc_body)
    return kern(proposals, featYX)


def _fc1_body(x_ref, w_ref, b_ref, out_ref):
    out_ref[...] = jnp.maximum(
        jnp.dot(x_ref[...], w_ref[...], preferred_element_type=jnp.float32)
        + b_ref[...], 0.0)


def _head_body(h1_ref, w2_ref, b2_ref, wc_ref, bc_ref, wr_ref, br_ref,
               props_ref, sc_ref, bx1_ref, by1_ref, bx2_ref, by2_ref):
    h2 = jnp.maximum(
        jnp.dot(h1_ref[...], w2_ref[...], preferred_element_type=jnp.float32)
        + b2_ref[...], 0.0)
    logits = jnp.dot(h2, wc_ref[...],
                     preferred_element_type=jnp.float32) + bc_ref[...]
    col = lax.broadcasted_iota(jnp.int32, (_N, 128), 1)
    logits = jnp.where(col < _NCLS, logits, -1e30)
    m = jnp.max(logits, axis=1, keepdims=True)
    e = jnp.exp(logits - m)
    sc_ref[...] = e / jnp.sum(e, axis=1, keepdims=True)

    reg = jnp.dot(h2, wr_ref[...],
                  preferred_element_type=jnp.float32) + br_ref[...]
    px1 = props_ref[:, 0:1]
    py1 = props_ref[:, 1:2]
    px2 = props_ref[:, 2:3]
    py2 = props_ref[:, 3:4]
    wp = px2 - px1
    hp = py2 - py1
    cxp = px1 + 0.5 * wp
    cyp = py1 + 0.5 * hp
    dx = reg[:, 0:128] / 10.0
    dy = reg[:, 128:256] / 10.0
    dw = jnp.minimum(reg[:, 256:384] / 5.0, _LOGMAX)
    dh = jnp.minimum(reg[:, 384:512] / 5.0, _LOGMAX)
    pcx = dx * wp + cxp
    pcy = dy * hp + cyp
    pw = jnp.exp(dw) * wp
    ph = jnp.exp(dh) * hp
    bx1_ref[...] = jnp.clip(pcx - 0.5 * pw, 0.0, _IMG)
    by1_ref[...] = jnp.clip(pcy - 0.5 * ph, 0.0, _IMG)
    bx2_ref[...] = jnp.clip(pcx + 0.5 * pw, 0.0, _IMG)
    by2_ref[...] = jnp.clip(pcy + 0.5 * ph, 0.0, _IMG)


def _nms_body(fs_ref, x1_ref, y1_ref, x2_ref, y2_ref, sel_ref,
              s_ref, ox1_ref, oy1_ref, ox2_ref, oy2_ref, aa_ref, done_ref):
    rq = lax.broadcasted_iota(jnp.int32, (_QROWS, 128), 0)
    cq = lax.broadcasted_iota(jnp.int32, (_QROWS, 128), 1)
    q = rq * 128 + cq
    lbl = (q % (_NCLS - 1) + 1).astype(jnp.float32)
    off = lbl * (_IMG + 1.0)
    x1 = x1_ref[...]
    y1 = y1_ref[...]
    x2 = x2_ref[...]
    y2 = y2_ref[...]
    fs = fs_ref[...]
    valid = ((fs > _SCORE_TH) & (x2 - x1 >= 0.01) & (y2 - y1 >= 0.01)
             & (q < _NQ))
    s_ref[...] = jnp.where(valid, fs, -1e30)
    ox1_ref[...] = x1 + off
    oy1_ref[...] = y1 + off
    ox2_ref[...] = x2 + off
    oy2_ref[...] = y2 + off
    aa_ref[...] = (ox2_ref[...] - ox1_ref[...]) * (oy2_ref[...] - oy1_ref[...])
    sel_ref[...] = jnp.zeros((128, 128), jnp.int32)
    done_ref[0] = 0

    def body(it, carry):
        @pl.when(done_ref[0] == 0)
        def _():
            s = s_ref[...]
            mval = jnp.max(s)
            jq = jnp.min(jnp.where(s == mval, q, jnp.int32(2**30)))
            onehot = q == jq
            b0x1 = jnp.sum(jnp.where(onehot, ox1_ref[...], 0.0))
            b0y1 = jnp.sum(jnp.where(onehot, oy1_ref[...], 0.0))
            b0x2 = jnp.sum(jnp.where(onehot, ox2_ref[...], 0.0))
            b0y2 = jnp.sum(jnp.where(onehot, oy2_ref[...], 0.0))
            lane = lax.broadcasted_iota(jnp.int32, (1, 128), 1)
            row = jnp.where(lane == 0, jq,
                            jnp.where(lane == 1, jnp.int32(1), 0))
            sel_ref[pl.ds(it, 1), :] = row
            xx1 = jnp.maximum(b0x1, ox1_ref[...])
            yy1 = jnp.maximum(b0y1, oy1_ref[...])
            xx2 = jnp.minimum(b0x2, ox2_ref[...])
            yy2 = jnp.minimum(b0y2, oy2_ref[...])
            inter = (jnp.maximum(xx2 - xx1, 0.0) * jnp.maximum(yy2 - yy1, 0.0))
            a0 = (b0x2 - b0x1) * (b0y2 - b0y1)
            iou = inter / (a0 + aa_ref[...] - inter + 1e-9)
            s_new = jnp.where(iou <= _NMS_TH, s, -1e30)
            s_ref[...] = s_new
            done_ref[0] = jnp.where(jnp.max(s_new) > -1e29, 0, 1)
        return carry

    lax.fori_loop(0, _DETS, body, 0)


def kernel(features, proposals, W_fc1, b_fc1, W_fc2, b_fc2, W_cls, b_cls,
           W_reg, b_reg):
    f32 = jnp.float32
    featYX = jnp.transpose(features, (1, 2, 0)).reshape(_HW * _HW, _C)
    Wc_pad = jnp.zeros((_REP, 128), f32).at[:, :_NCLS].set(W_cls)
    bc_pad = jnp.zeros((1, 128), f32).at[0, :_NCLS].set(b_cls)
    cls_idx = jnp.arange(_NCLS)
    Wr_pad = jnp.zeros((_REP, 512), f32)
    br_pad = jnp.zeros((1, 512), f32)
    for g in range(4):
        Wr_pad = Wr_pad.at[:, g * 128 + cls_idx].set(W_reg[:, g::4])
        br_pad = br_pad.at[0, g * 128 + cls_idx].set(b_reg[g::4])

    props4 = jnp.pad(proposals.T, ((0, 0), (0, 24)))
    rf = _roi_sc(props4, featYX)
    rf2 = rf.reshape(1024, _OUT * _OUT, _C).transpose(0, 2, 1).reshape(
        1024, _OUT * _OUT * _C)[:_N]

    NB = 256
    h1 = pl.pallas_call(
        _fc1_body,
        grid=(_REP // NB,),
        in_specs=[
            pl.BlockSpec((_N, _OUT * _OUT * _C), lambda n: (0, 0)),
            pl.BlockSpec((_OUT * _OUT * _C, NB), lambda n: (0, n)),
            pl.BlockSpec((1, NB), lambda n: (0, n)),
        ],
        out_specs=pl.BlockSpec((_N, NB), lambda n: (0, n)),
        out_shape=jax.ShapeDtypeStruct((_N, _REP), f32),
        compiler_params=pltpu.CompilerParams(
            dimension_semantics=("arbitrary",)),
    )(rf2, W_fc1, b_fc1.reshape(1, _REP))

    sc, bx1, by1, bx2, by2 = pl.pallas_call(
        _head_body,
        in_specs=[
            pl.BlockSpec((_N, _REP), lambda: (0, 0)),
            pl.BlockSpec((_REP, _REP), lambda: (0, 0)),
            pl.BlockSpec((1, _REP), lambda: (0, 0)),
            pl.BlockSpec((_REP, 128), lambda: (0, 0)),
            pl.BlockSpec((1, 128), lambda: (0, 0)),
            pl.BlockSpec((_REP, 512), lambda: (0, 0)),
            pl.BlockSpec((1, 512), lambda: (0, 0)),
            pl.BlockSpec((_N, 4), lambda: (0, 0)),
        ],
        out_specs=[pl.BlockSpec((_N, 128), lambda: (0, 0))] * 5,
        out_shape=[jax.ShapeDtypeStruct((_N, 128), f32)] * 5,
    )(h1, W_fc2, b_fc2.reshape(1, _REP), Wc_pad, bc_pad, Wr_pad, br_pad,
      proposals)

    def flat_pad(a, padval):
        v = a[:, 1:_NCLS].reshape(-1)
        v = jnp.concatenate([v, jnp.full((_NQP - _NQ,), padval, f32)])
        return v.reshape(_QROWS, 128)

    fsq = flat_pad(sc, 0.0)
    fx1 = flat_pad(bx1, 0.0)
    fy1 = flat_pad(by1, 0.0)
    fx2 = flat_pad(bx2, 0.0)
    fy2 = flat_pad(by2, 0.0)

    sel = pl.pallas_call(
        _nms_body,
        in_specs=[pl.BlockSpec((_QROWS, 128), lambda: (0, 0))] * 5,
        out_specs=pl.BlockSpec((128, 128), lambda: (0, 0)),
        out_shape=jax.ShapeDtypeStruct((128, 128), jnp.int32),
        scratch_shapes=[
            pltpu.VMEM((_QROWS, 128), f32),
            pltpu.VMEM((_QROWS, 128), f32),
            pltpu.VMEM((_QROWS, 128), f32),
            pltpu.VMEM((_QROWS, 128), f32),
            pltpu.VMEM((_QROWS, 128), f32),
            pltpu.VMEM((_QROWS, 128), f32),
            pltpu.SMEM((1,), jnp.int32),
        ],
    )(fsq, fx1, fy1, fx2, fy2)

    fin = sel[:_DETS, 0]
    finv = sel[:_DETS, 1] > 0
    fb = jnp.stack([fx1.reshape(-1), fy1.reshape(-1),
                    fx2.reshape(-1), fy2.reshape(-1)], axis=1)
    fs = fsq.reshape(-1)
    boxes = jnp.where(finv[:, None], fb[fin], 0.0)
    scores = jnp.where(finv, fs[fin], 0.0)
    labels = jnp.where(finv, (fin % (_NCLS - 1) + 1).astype(jnp.int32), 0)
    return boxes, scores, labels


# speedup vs baseline: 3.9184x; 1.0011x over previous
import functools
import math

import jax
import jax.numpy as jnp
from jax import lax
from jax.experimental import pallas as pl
from jax.experimental.pallas import tpu as pltpu
from jax.experimental.pallas import tpu_sc as plsc

_N = 1000
_C = 128
_HW = 50
_OUT = 7
_SCALE = 1.0 / 16.0
_NCLS = 91
_REP = 1024
_SCORE_TH = 0.05
_NMS_TH = 0.5
_DETS = 100
_IMG = 800.0
_LOGMAX = math.log(1000.0 / 16.0)

_NQ = _N * (_NCLS - 1)
_NQP = 90112
_QROWS = _NQP // 128


_NW = 32
_TPW = 1568
_CH = 112
_NCH = _TPW // _CH


def _roi_sc_body(props_hbm, feat_hbm, out_hbm, x1v, y1v, x2v, y2v,
                 i00, i01, i10, i11, lyv, lxv,
                 f00, f01, f10, f11, outv, sem):
    wid = lax.axis_index("s") * 2 + lax.axis_index("c")
    pltpu.sync_copy(props_hbm.at[0], x1v)
    pltpu.sync_copy(props_hbm.at[1], y1v)
    pltpu.sync_copy(props_hbm.at[2], x2v)
    pltpu.sync_copy(props_hbm.at[3], y2v)
    lanes = lax.broadcasted_iota(jnp.int32, (16,), 0)
    zc = jnp.zeros((16,), jnp.int32)

    def chunk_body(ch, _):
        base = wid * _TPW + ch * _CH

        def param_body(s, _):
            t = base + s * 16 + lanes
            tf = t.astype(jnp.float32) + 0.5
            p = jnp.minimum((tf * (1.0 / 49.0)).astype(jnp.int32), _N - 1)
            ij = t - p * 49
            ijf = ij.astype(jnp.float32) + 0.5
            iq = (ijf * (1.0 / 7.0)).astype(jnp.int32)
            i_ = iq.astype(jnp.float32)
            j_ = (ij - iq * 7).astype(jnp.float32)
            x1 = plsc.load_gather(x1v, [p]) * _SCALE
            y1 = plsc.load_gather(y1v, [p]) * _SCALE
            x2 = plsc.load_gather(x2v, [p]) * _SCALE
            y2 = plsc.load_gather(y2v, [p]) * _SCALE
            bw = jnp.maximum(x2 - x1, 1.0) / _OUT
            bh = jnp.maximum(y2 - y1, 1.0) / _OUT
            gx = jnp.clip(x1 + bw * (j_ + 0.5), 0.0, _HW - 1.0)
            gy = jnp.clip(y1 + bh * (i_ + 0.5), 0.0, _HW - 1.0)
            x0 = gx.astype(jnp.int32)
            y0 = gy.astype(jnp.int32)
            lx = gx - x0.astype(jnp.float32)
            ly = gy - y0.astype(jnp.float32)
            x1i = jnp.minimum(x0 + 1, _HW - 1)
            y1i = jnp.minimum(y0 + 1, _HW - 1)
            sl = pl.ds(s * 16, 16)
            i00[sl] = y0 * _HW + x0
            i01[sl] = y0 * _HW + x1i
            i10[sl] = y1i * _HW + x0
            i11[sl] = y1i * _HW + x1i
            lyv[sl] = ly
            lxv[sl] = lx
            return 0

        lax.fori_loop(0, _CH // 16, param_body, 0)
        cp0 = pltpu.async_copy(feat_hbm.at[i00], f00, sem)
        cp1 = pltpu.async_copy(feat_hbm.at[i01], f01, sem)
        cp2 = pltpu.async_copy(feat_hbm.at[i10], f10, sem)
        cp3 = pltpu.async_copy(feat_hbm.at[i11], f11, sem)
        cp0.wait()
        cp1.wait()
        cp2.wait()
        cp3.wait()

        def pt_body(pt, _):
            ib = zc + pt
            ly = plsc.load_gather(lyv, [ib])
            lx = plsc.load_gather(lxv, [ib])
            a = 1.0 - ly
            b = 1.0 - lx
            for cs in range(8):
                sl = pl.ds(cs * 16, 16)
                v = (f00[pt, sl] * a * b + f01[pt, sl] * a * lx
                     + f10[pt, sl] * ly * b + f11[pt, sl] * ly * lx)
                outv[pt, sl] = v
            return 0

        lax.fori_loop(0, _CH, pt_body, 0)
        pltpu.sync_copy(outv, out_hbm.at[pl.ds(base, _CH)])
        return 0

    lax.fori_loop(0, _NCH, chunk_body, 0)


def _roi_sc(proposals, featYX):
    f32 = jnp.float32
    kern = functools.partial(
        pl.kernel,
        out_type=jax.ShapeDtypeStruct((_NW * _TPW, _C), f32),
        mesh=plsc.VectorSubcoreMesh(core_axis_name="c", subcore_axis_name="s"),
        compiler_params=pltpu.CompilerParams(needs_layout_passes=False),
        scratch_types=[
            pltpu.VMEM((1024,), f32),
            pltpu.VMEM((1024,), f32),
            pltpu.VMEM((1024,), f32),
            pltpu.VMEM((1024,), f32),
            pltpu.VMEM((_CH,), jnp.int32),
            pltpu.VMEM((_CH,), jnp.int32),
            pltpu.VMEM((_CH,), jnp.int32),
            pltpu.VMEM((_CH,), jnp.int32),
            pltpu.VMEM((_CH,), f32),
            pltpu.VMEM((_CH,), f32),
            pltpu.VMEM((_CH, _C), f32),
            pltpu.VMEM((_CH, _C), f32),
            pltpu.VMEM((_CH, _C), f32),
            pltpu.VMEM((_CH, _C), f32),
            pltpu.VMEM((_CH, _C), f32),
            pltpu.SemaphoreType.DMA,
        ],
    )(_roi_sc_body)
    return kern(proposals, featYX)


def _fc1_body(x_ref, w_ref, b_ref, out_ref):
    out_ref[...] = jnp.maximum(
        jnp.dot(x_ref[...], w_ref[...], preferred_element_type=jnp.float32)
        + b_ref[...], 0.0)


def _head_body(h1_ref, w2_ref, b2_ref, wc_ref, bc_ref, wr_ref, br_ref,
               props_ref, sc_ref, bx1_ref, by1_ref, bx2_ref, by2_ref):
    h2 = jnp.maximum(
        jnp.dot(h1_ref[...], w2_ref[...], preferred_element_type=jnp.float32)
        + b2_ref[...], 0.0)
    logits = jnp.dot(h2, wc_ref[...],
                     preferred_element_type=jnp.float32) + bc_ref[...]
    col = lax.broadcasted_iota(jnp.int32, (_N, 128), 1)
    logits = jnp.where(col < _NCLS, logits, -1e30)
    m = jnp.max(logits, axis=1, keepdims=True)
    e = jnp.exp(logits - m)
    sc_ref[...] = e / jnp.sum(e, axis=1, keepdims=True)

    reg = jnp.dot(h2, wr_ref[...],
                  preferred_element_type=jnp.float32) + br_ref[...]
    px1 = props_ref[:, 0:1]
    py1 = props_ref[:, 1:2]
    px2 = props_ref[:, 2:3]
    py2 = props_ref[:, 3:4]
    wp = px2 - px1
    hp = py2 - py1
    cxp = px1 + 0.5 * wp
    cyp = py1 + 0.5 * hp
    dx = reg[:, 0:128] / 10.0
    dy = reg[:, 128:256] / 10.0
    dw = jnp.minimum(reg[:, 256:384] / 5.0, _LOGMAX)
    dh = jnp.minimum(reg[:, 384:512] / 5.0, _LOGMAX)
    pcx = dx * wp + cxp
    pcy = dy * hp + cyp
    pw = jnp.exp(dw) * wp
    ph = jnp.exp(dh) * hp
    bx1_ref[...] = jnp.clip(pcx - 0.5 * pw, 0.0, _IMG)
    by1_ref[...] = jnp.clip(pcy - 0.5 * ph, 0.0, _IMG)
    bx2_ref[...] = jnp.clip(pcx + 0.5 * pw, 0.0, _IMG)
    by2_ref[...] = jnp.clip(pcy + 0.5 * ph, 0.0, _IMG)


def _nms_body(fs_ref, x1_ref, y1_ref, x2_ref, y2_ref, sel_ref,
              s_ref, ox1_ref, oy1_ref, ox2_ref, oy2_ref, aa_ref, done_ref):
    rq = lax.broadcasted_iota(jnp.int32, (_QROWS, 128), 0)
    cq = lax.broadcasted_iota(jnp.int32, (_QROWS, 128), 1)
    q = rq * 128 + cq
    lbl = (q % (_NCLS - 1) + 1).astype(jnp.float32)
    off = lbl * (_IMG + 1.0)
    x1 = x1_ref[...]
    y1 = y1_ref[...]
    x2 = x2_ref[...]
    y2 = y2_ref[...]
    fs = fs_ref[...]
    valid = ((fs > _SCORE_TH) & (x2 - x1 >= 0.01) & (y2 - y1 >= 0.01)
             & (q < _NQ))
    s_ref[...] = jnp.where(valid, fs, -1e30)
    ox1_ref[...] = x1 + off
    oy1_ref[...] = y1 + off
    ox2_ref[...] = x2 + off
    oy2_ref[...] = y2 + off
    aa_ref[...] = (ox2_ref[...] - ox1_ref[...]) * (oy2_ref[...] - oy1_ref[...])
    sel_ref[...] = jnp.zeros((128, 128), jnp.int32)
    done_ref[0] = 0

    def body(it, carry):
        @pl.when(done_ref[0] == 0)
        def _():
            s = s_ref[...]
            mval = jnp.max(s)
            jq = jnp.min(jnp.where(s == mval, q, jnp.int32(2**30)))
            onehot = q == jq
            b0x1 = jnp.sum(jnp.where(onehot, ox1_ref[...], 0.0))
            b0y1 = jnp.sum(jnp.where(onehot, oy1_ref[...], 0.0))
            b0x2 = jnp.sum(jnp.where(onehot, ox2_ref[...], 0.0))
            b0y2 = jnp.sum(jnp.where(onehot, oy2_ref[...], 0.0))
            lane = lax.broadcasted_iota(jnp.int32, (1, 128), 1)
            row = jnp.where(lane == 0, jq,
                            jnp.where(lane == 1, jnp.int32(1), 0))
            sel_ref[pl.ds(it, 1), :] = row
            xx1 = jnp.maximum(b0x1, ox1_ref[...])
            yy1 = jnp.maximum(b0y1, oy1_ref[...])
            xx2 = jnp.minimum(b0x2, ox2_ref[...])
            yy2 = jnp.minimum(b0y2, oy2_ref[...])
            inter = (jnp.maximum(xx2 - xx1, 0.0) * jnp.maximum(yy2 - yy1, 0.0))
            a0 = (b0x2 - b0x1) * (b0y2 - b0y1)
            iou = inter / (a0 + aa_ref[...] - inter + 1e-9)
            s_new = jnp.where(iou <= _NMS_TH, s, -1e30)
            s_ref[...] = s_new
            done_ref[0] = jnp.where(jnp.max(s_new) > -1e29, 0, 1)
        return carry

    lax.fori_loop(0, _DETS, body, 0)


def kernel(features, proposals, W_fc1, b_fc1, W_fc2, b_fc2, W_cls, b_cls,
           W_reg, b_reg):
    f32 = jnp.float32
    featYX = jnp.transpose(features, (1, 2, 0)).reshape(_HW * _HW, _C)
    Wc_pad = jnp.zeros((_REP, 128), f32).at[:, :_NCLS].set(W_cls)
    bc_pad = jnp.zeros((1, 128), f32).at[0, :_NCLS].set(b_cls)
    cls_idx = jnp.arange(_NCLS)
    Wr_pad = jnp.zeros((_REP, 512), f32)
    br_pad = jnp.zeros((1, 512), f32)
    for g in range(4):
        Wr_pad = Wr_pad.at[:, g * 128 + cls_idx].set(W_reg[:, g::4])
        br_pad = br_pad.at[0, g * 128 + cls_idx].set(b_reg[g::4])

    props4 = jnp.pad(proposals.T, ((0, 0), (0, 24)))
    rf = _roi_sc(props4, featYX)
    rf2 = rf.reshape(1024, _OUT * _OUT, _C).transpose(0, 2, 1).reshape(
        1024, _OUT * _OUT * _C)[:_N]

    NB = 256
    h1 = pl.pallas_call(
        _fc1_body,
        grid=(_REP // NB,),
        in_specs=[
            pl.BlockSpec((_N, _OUT * _OUT * _C), lambda n: (0, 0)),
            pl.BlockSpec((_OUT * _OUT * _C, NB), lambda n: (0, n)),
            pl.BlockSpec((1, NB), lambda n: (0, n)),
        ],
        out_specs=pl.BlockSpec((_N, NB), lambda n: (0, n)),
        out_shape=jax.ShapeDtypeStruct((_N, _REP), f32),
        compiler_params=pltpu.CompilerParams(
            dimension_semantics=("arbitrary",)),
    )(rf2, W_fc1, b_fc1.reshape(1, _REP))

    sc, bx1, by1, bx2, by2 = pl.pallas_call(
        _head_body,
        in_specs=[
            pl.BlockSpec((_N, _REP), lambda: (0, 0)),
            pl.BlockSpec((_REP, _REP), lambda: (0, 0)),
            pl.BlockSpec((1, _REP), lambda: (0, 0)),
            pl.BlockSpec((_REP, 128), lambda: (0, 0)),
            pl.BlockSpec((1, 128), lambda: (0, 0)),
            pl.BlockSpec((_REP, 512), lambda: (0, 0)),
            pl.BlockSpec((1, 512), lambda: (0, 0)),
            pl.BlockSpec((_N, 4), lambda: (0, 0)),
        ],
        out_specs=[pl.BlockSpec((_N, 128), lambda: (0, 0))] * 5,
        out_shape=[jax.ShapeDtypeStruct((_N, 128), f32)] * 5,
    )(h1, W_fc2, b_fc2.reshape(1, _REP), Wc_pad, bc_pad, Wr_pad, br_pad,
      proposals)

    def flat_pad(a, padval):
        v = a[:, 1:_NCLS].reshape(-1)
        v = jnp.concatenate([v, jnp.full((_NQP - _NQ,), padval, f32)])
        return v.reshape(_QROWS, 128)

    fsq = flat_pad(sc, 0.0)
    fx1 = flat_pad(bx1, 0.0)
    fy1 = flat_pad(by1, 0.0)
    fx2 = flat_pad(bx2, 0.0)
    fy2 = flat_pad(by2, 0.0)

    sel = pl.pallas_call(
        _nms_body,
        in_specs=[pl.BlockSpec((_QROWS, 128), lambda: (0, 0))] * 5,
        out_specs=pl.BlockSpec((128, 128), lambda: (0, 0)),
        out_shape=jax.ShapeDtypeStruct((128, 128), jnp.int32),
        scratch_shapes=[
            pltpu.VMEM((_QROWS, 128), f32),
            pltpu.VMEM((_QROWS, 128), f32),
            pltpu.VMEM((_QROWS, 128), f32),
            pltpu.VMEM((_QROWS, 128), f32),
            pltpu.VMEM((_QROWS, 128), f32),
            pltpu.VMEM((_QROWS, 128), f32),
            pltpu.SMEM((1,), jnp.int32),
        ],
    )(fsq, fx1, fy1, fx2, fy2)

    fin = sel[:_DETS, 0]
    finv = sel[:_DETS, 1] > 0
    fb = jnp.stack([fx1.reshape(-1), fy1.reshape(-1),
                    fx2.reshape(-1), fy2.reshape(-1)], axis=1)
    fs = fsq.reshape(-1)
    boxes = jnp.where(finv[:, None], fb[fin], 0.0)
    scores = jnp.where(finv, fs[fin], 0.0)
    labels = jnp.where(finv, (fin % (_NCLS - 1) + 1).astype(jnp.int32), 0)
    return boxes, scores, labels
